# Initial kernel scaffold; baseline (speedup 1.0000x reference)
import jax
import jax.numpy as jnp
from jax.experimental import pallas as pl


def kernel(x, Wqkv, bqkv, rpb_table, Wproj, bproj):
    raise NotImplementedError("write your pallas kernel here")


# speedup vs baseline: 16.9447x; 16.9447x over previous
import numpy as np
import jax
import jax.numpy as jnp
from jax.experimental import pallas as pl
from jax.experimental.pallas import tpu as pltpu

_B = 1024
_WIN = 8
_N = _WIN * _WIN
_DIM = 384
_NH = 12
_HD = _DIM // _NH
_K = max(1, int(_N * 0.25))
_WB = 4
_T = _WB * _N
_SCALE = _HD ** (-0.5)
_NEG = -1e30


def _rel_pos_index(w: int) -> np.ndarray:
    coords = np.stack(np.meshgrid(np.arange(w), np.arange(w), indexing='ij'))
    cf = coords.reshape(2, -1)
    rel = cf[:, :, None] - cf[:, None, :]
    rel = rel.transpose(1, 2, 0).copy()
    rel[:, :, 0] += w - 1
    rel[:, :, 1] += w - 1
    rel[:, :, 0] *= 2 * w - 1
    return rel.sum(-1)


def _body(x_ref, wqkv_ref, bqkv_ref, biast_ref, wproj_ref, bproj_ref, o_ref,
          qkv_scr, acct_scr):
    xb = x_ref[...]
    qkv_scr[...] = (
        jnp.dot(xb, wqkv_ref[...], preferred_element_type=jnp.float32)
        + bqkv_ref[...])
    acct_scr[...] = jnp.zeros((_WB, _DIM, _N), jnp.float32)

    lane = jax.lax.broadcasted_iota(jnp.int32, (_N, 128), 1)

    for h in range(_NH):
        r = (h % 4) * _HD
        gq = h // 4
        gk = 3 + h // 4
        gv = 6 + h // 4
        m32 = (lane >= r) & (lane < r + _HD)
        biast_h = biast_ref[h]
        for w in range(_WB):
            rs = w * _N
            qg = qkv_scr[pl.ds(rs, _N), pl.ds(gq * 128, 128)]
            kg = qkv_scr[pl.ds(rs, _N), pl.ds(gk * 128, 128)]
            vg = qkv_scr[pl.ds(rs, _N), pl.ds(gv * 128, 128)]
            q = jnp.where(m32, qg, 0.0)
            v = jnp.where(m32, vg, 0.0)
            st = jax.lax.dot_general(kg, q, (((1,), (1,)), ((), ())),
                                     preferred_element_type=jnp.float32)
            st = st + biast_h
            work = st
            m0 = jnp.max(work, axis=0, keepdims=True)
            m = m0
            for _ in range(_K - 1):
                work = jnp.where(work >= m, _NEG, work)
                m = jnp.max(work, axis=0, keepdims=True)
            p = jnp.where(st >= m, jnp.exp(st - m0), 0.0)
            p = p * (1.0 / jnp.sum(p, axis=0, keepdims=True))
            outt = jax.lax.dot_general(v, p, (((0,), (0,)), ((), ())),
                                       preferred_element_type=jnp.float32)
            acct_scr[w, pl.ds(gv * 128 - 2 * _DIM, 128), :] += outt

    for w in range(_WB):
        yw = jax.lax.dot_general(acct_scr[w], wproj_ref[...],
                                 (((0,), (0,)), ((), ())),
                                 preferred_element_type=jnp.float32)
        o_ref[pl.ds(w * _N, _N), :] = yw + bproj_ref[...]


def kernel(x, Wqkv, bqkv, rpb_table, Wproj, bproj):
    x2 = x.reshape(_B * _N, _DIM)
    rel = _rel_pos_index(_WIN).reshape(-1)
    bias = rpb_table[rel].reshape(_N, _N, _NH)
    biast = jnp.transpose(bias, (2, 1, 0))
    qscale = jnp.concatenate(
        [jnp.full((_DIM,), _SCALE, jnp.float32),
         jnp.ones((2 * _DIM,), jnp.float32)])
    Wqkv = Wqkv * qscale
    bqkv = bqkv * qscale

    out = pl.pallas_call(
        _body,
        grid=(_B // _WB,),
        in_specs=[
            pl.BlockSpec((_T, _DIM), lambda i: (i, 0)),
            pl.BlockSpec((_DIM, 3 * _DIM), lambda i: (0, 0)),
            pl.BlockSpec((1, 3 * _DIM), lambda i: (0, 0)),
            pl.BlockSpec((_NH, _N, _N), lambda i: (0, 0, 0)),
            pl.BlockSpec((_DIM, _DIM), lambda i: (0, 0)),
            pl.BlockSpec((1, _DIM), lambda i: (0, 0)),
        ],
        out_specs=pl.BlockSpec((_T, _DIM), lambda i: (i, 0)),
        out_shape=jax.ShapeDtypeStruct((_B * _N, _DIM), jnp.float32),
        scratch_shapes=[
            pltpu.VMEM((_T, 3 * _DIM), jnp.float32),
            pltpu.VMEM((_WB, _DIM, _N), jnp.float32),
        ],
    )(x2, Wqkv, bqkv.reshape(1, -1), biast, Wproj, bproj.reshape(1, -1))
    return out.reshape(_B, _N, _DIM)


# speedup vs baseline: 18.7877x; 1.1088x over previous
import numpy as np
import jax
import jax.numpy as jnp
from jax.experimental import pallas as pl
from jax.experimental.pallas import tpu as pltpu

_B = 1024
_WIN = 8
_N = _WIN * _WIN
_DIM = 384
_NH = 12
_HD = _DIM // _NH
_K = max(1, int(_N * 0.25))
_WB = 4
_T = _WB * _N
_SCALE = _HD ** (-0.5)
_NEG = -1e30


def _rel_pos_index(w: int) -> np.ndarray:
    coords = np.stack(np.meshgrid(np.arange(w), np.arange(w), indexing='ij'))
    cf = coords.reshape(2, -1)
    rel = cf[:, :, None] - cf[:, None, :]
    rel = rel.transpose(1, 2, 0).copy()
    rel[:, :, 0] += w - 1
    rel[:, :, 1] += w - 1
    rel[:, :, 0] *= 2 * w - 1
    return rel.sum(-1)


def _vmax_tree(xv):
    r = jnp.maximum(
        jnp.maximum(jnp.maximum(xv[0:8], xv[8:16]),
                    jnp.maximum(xv[16:24], xv[24:32])),
        jnp.maximum(jnp.maximum(xv[32:40], xv[40:48]),
                    jnp.maximum(xv[48:56], xv[56:64])))
    r = jnp.maximum(r[0:4], r[4:8])
    r = jnp.maximum(r[0:2], r[2:4])
    return jnp.maximum(r[0:1], r[1:2])


def _vsum_tree(xv):
    r = (((xv[0:8] + xv[8:16]) + (xv[16:24] + xv[24:32]))
         + ((xv[32:40] + xv[40:48]) + (xv[48:56] + xv[56:64])))
    r = r[0:4] + r[4:8]
    r = r[0:2] + r[2:4]
    return r[0:1] + r[1:2]


def _body(x_ref, wqkv_ref, bqkv_ref, biast_ref, wproj_ref, bproj_ref, o_ref,
          qkv_scr, acct_scr):
    xb = x_ref[...]
    qkv_scr[...] = (
        jnp.dot(xb, wqkv_ref[...], preferred_element_type=jnp.float32)
        + bqkv_ref[...])
    acct_scr[...] = jnp.zeros((_WB, _DIM, _N), jnp.float32)

    lane = jax.lax.broadcasted_iota(jnp.int32, (_N, 128), 1)

    for h in range(_NH):
        r = (h % 4) * _HD
        gq = h // 4
        gk = 3 + h // 4
        gv = 6 + h // 4
        m32 = (lane >= r) & (lane < r + _HD)
        biast_h = biast_ref[h]
        for w in range(_WB):
            rs = w * _N
            qg = qkv_scr[pl.ds(rs, _N), pl.ds(gq * 128, 128)]
            kg = qkv_scr[pl.ds(rs, _N), pl.ds(gk * 128, 128)]
            vg = qkv_scr[pl.ds(rs, _N), pl.ds(gv * 128, 128)]
            q = jnp.where(m32, qg, 0.0) * _SCALE
            v = jnp.where(m32, vg, 0.0)
            s_qk = jax.lax.dot_general(q, kg, (((1,), (1,)), ((), ())),
                                       preferred_element_type=jnp.float32)
            st = jnp.transpose(s_qk) + biast_h
            work = st
            m0 = _vmax_tree(work)
            m = m0
            for _ in range(_K - 1):
                work = jnp.where(work >= m, _NEG, work)
                m = _vmax_tree(work)
            p = jnp.where(st >= m, jnp.exp(st - m0), 0.0)
            p = p * (1.0 / _vsum_tree(p))
            outt = jax.lax.dot_general(v, p, (((0,), (0,)), ((), ())),
                                       preferred_element_type=jnp.float32)
            acct_scr[w, pl.ds(gv * 128 - 2 * _DIM, 128), :] += outt

    for w in range(_WB):
        yw = jax.lax.dot_general(acct_scr[w], wproj_ref[...],
                                 (((0,), (0,)), ((), ())),
                                 preferred_element_type=jnp.float32)
        o_ref[pl.ds(w * _N, _N), :] = yw + bproj_ref[...]


def kernel(x, Wqkv, bqkv, rpb_table, Wproj, bproj):
    x2 = x.reshape(_B * _N, _DIM)
    rel = _rel_pos_index(_WIN).reshape(-1)
    bias = rpb_table[rel].reshape(_N, _N, _NH)
    biast = jnp.transpose(bias, (2, 1, 0))

    out = pl.pallas_call(
        _body,
        grid=(_B // _WB,),
        in_specs=[
            pl.BlockSpec((_T, _DIM), lambda i: (i, 0)),
            pl.BlockSpec((_DIM, 3 * _DIM), lambda i: (0, 0)),
            pl.BlockSpec((1, 3 * _DIM), lambda i: (0, 0)),
            pl.BlockSpec((_NH, _N, _N), lambda i: (0, 0, 0)),
            pl.BlockSpec((_DIM, _DIM), lambda i: (0, 0)),
            pl.BlockSpec((1, _DIM), lambda i: (0, 0)),
        ],
        out_specs=pl.BlockSpec((_T, _DIM), lambda i: (i, 0)),
        out_shape=jax.ShapeDtypeStruct((_B * _N, _DIM), jnp.float32),
        scratch_shapes=[
            pltpu.VMEM((_T, 3 * _DIM), jnp.float32),
            pltpu.VMEM((_WB, _DIM, _N), jnp.float32),
        ],
    )(x2, Wqkv, bqkv.reshape(1, -1), biast, Wproj, bproj.reshape(1, -1))
    return out.reshape(_B, _N, _DIM)


# speedup vs baseline: 39.8696x; 2.1221x over previous
import numpy as np
import jax
import jax.numpy as jnp
from jax.experimental import pallas as pl
from jax.experimental.pallas import tpu as pltpu

_B = 1024
_WIN = 8
_N = _WIN * _WIN
_DIM = 384
_NH = 12
_HD = _DIM // _NH
_K = max(1, int(_N * 0.25))
_WB = 4
_T = _WB * _N
_SCALE = _HD ** (-0.5)
_NEG = -1e30


def _rel_pos_index(w: int) -> np.ndarray:
    coords = np.stack(np.meshgrid(np.arange(w), np.arange(w), indexing='ij'))
    cf = coords.reshape(2, -1)
    rel = cf[:, :, None] - cf[:, None, :]
    rel = rel.transpose(1, 2, 0).copy()
    rel[:, :, 0] += w - 1
    rel[:, :, 1] += w - 1
    rel[:, :, 0] *= 2 * w - 1
    return rel.sum(-1)


def _vmax_tree(xv):
    r = jnp.maximum(
        jnp.maximum(jnp.maximum(xv[0:8], xv[8:16]),
                    jnp.maximum(xv[16:24], xv[24:32])),
        jnp.maximum(jnp.maximum(xv[32:40], xv[40:48]),
                    jnp.maximum(xv[48:56], xv[56:64])))
    r = jnp.maximum(r[0:4], r[4:8])
    r = jnp.maximum(r[0:2], r[2:4])
    return jnp.maximum(r[0:1], r[1:2])


def _vsum_tree(xv):
    r = (((xv[0:8] + xv[8:16]) + (xv[16:24] + xv[24:32]))
         + ((xv[32:40] + xv[40:48]) + (xv[48:56] + xv[56:64])))
    r = r[0:4] + r[4:8]
    r = r[0:2] + r[2:4]
    return r[0:1] + r[1:2]


def _body(x_ref, wqkv_ref, bqkv_ref, biast2_ref, wproj_ref, bproj_ref, o_ref,
          qkv_scr, acc_scr):
    xb = x_ref[...]
    qkv_scr[...] = (
        jnp.dot(xb, wqkv_ref[...], preferred_element_type=jnp.float32)
        + bqkv_ref[...])
    acc_scr[...] = jnp.zeros((_T, _DIM), jnp.float32)

    lane = jax.lax.broadcasted_iota(jnp.int32, (_N, 128), 1)

    for g in range(3):
        for cp in range(2):
            c1, c2 = 2 * cp, 2 * cp + 1
            m1 = (lane >= c1 * _HD) & (lane < (c1 + 1) * _HD)
            m2 = (lane >= c2 * _HD) & (lane < (c2 + 1) * _HD)
            biast2_h = biast2_ref[2 * g + cp]
            for w in range(_WB):
                rs = w * _N
                qg = qkv_scr[pl.ds(rs, _N), pl.ds(g * 128, 128)]
                kg = qkv_scr[pl.ds(rs, _N), pl.ds((3 + g) * 128, 128)]
                vg = qkv_scr[pl.ds(rs, _N), pl.ds((6 + g) * 128, 128)]
                q2 = jnp.concatenate(
                    [jnp.where(m1, qg, 0.0) * _SCALE,
                     jnp.where(m2, qg, 0.0) * _SCALE], axis=0)
                s2 = jax.lax.dot_general(q2, kg, (((1,), (1,)), ((), ())),
                                         preferred_element_type=jnp.float32)
                st = jnp.transpose(s2) + biast2_h
                work = st
                m0 = _vmax_tree(work)
                m = m0
                for _ in range(_K - 1):
                    work = jnp.where(work >= m, _NEG, work)
                    m = _vmax_tree(work)
                p = jnp.where(st >= m, jnp.exp(st - m0), 0.0)
                p = p * (1.0 / _vsum_tree(p))
                pt = jnp.transpose(p)
                for msk, pr in ((m1, pt[0:_N]), (m2, pt[_N:])):
                    v = jnp.where(msk, vg, 0.0)
                    out = jax.lax.dot_general(
                        pr, v, (((1,), (0,)), ((), ())),
                        preferred_element_type=jnp.float32)
                    acc_scr[pl.ds(rs, _N), pl.ds(g * 128, 128)] += out

    o_ref[...] = (
        jnp.dot(acc_scr[...], wproj_ref[...],
                preferred_element_type=jnp.float32)
        + bproj_ref[...])


def kernel(x, Wqkv, bqkv, rpb_table, Wproj, bproj):
    x2 = x.reshape(_B * _N, _DIM)
    rel = _rel_pos_index(_WIN).reshape(-1)
    bias = rpb_table[rel].reshape(_N, _N, _NH)
    biast = jnp.transpose(bias, (2, 1, 0))
    biast2 = biast.reshape(_NH // 2, 2, _N, _N)
    biast2 = jnp.transpose(biast2, (0, 2, 1, 3)).reshape(
        _NH // 2, _N, 2 * _N)

    out = pl.pallas_call(
        _body,
        grid=(_B // _WB,),
        in_specs=[
            pl.BlockSpec((_T, _DIM), lambda i: (i, 0)),
            pl.BlockSpec((_DIM, 3 * _DIM), lambda i: (0, 0)),
            pl.BlockSpec((1, 3 * _DIM), lambda i: (0, 0)),
            pl.BlockSpec((_NH // 2, _N, 2 * _N), lambda i: (0, 0, 0)),
            pl.BlockSpec((_DIM, _DIM), lambda i: (0, 0)),
            pl.BlockSpec((1, _DIM), lambda i: (0, 0)),
        ],
        out_specs=pl.BlockSpec((_T, _DIM), lambda i: (i, 0)),
        out_shape=jax.ShapeDtypeStruct((_B * _N, _DIM), jnp.float32),
        scratch_shapes=[
            pltpu.VMEM((_T, 3 * _DIM), jnp.float32),
            pltpu.VMEM((_T, _DIM), jnp.float32),
        ],
    )(x2, Wqkv, bqkv.reshape(1, -1), biast2, Wproj, bproj.reshape(1, -1))
    return out.reshape(_B, _N, _DIM)


# speedup vs baseline: 40.5560x; 1.0172x over previous
import numpy as np
import jax
import jax.numpy as jnp
from jax.experimental import pallas as pl
from jax.experimental.pallas import tpu as pltpu

_B = 1024
_WIN = 8
_N = _WIN * _WIN
_DIM = 384
_NH = 12
_HD = _DIM // _NH
_K = max(1, int(_N * 0.25))
_WB = 4
_T = _WB * _N
_SCALE = _HD ** (-0.5)
_NEG = -1e30


def _rel_pos_index(w: int) -> np.ndarray:
    coords = np.stack(np.meshgrid(np.arange(w), np.arange(w), indexing='ij'))
    cf = coords.reshape(2, -1)
    rel = cf[:, :, None] - cf[:, None, :]
    rel = rel.transpose(1, 2, 0).copy()
    rel[:, :, 0] += w - 1
    rel[:, :, 1] += w - 1
    rel[:, :, 0] *= 2 * w - 1
    return rel.sum(-1)


def _vmax_tree32(xv):
    r = jnp.maximum(jnp.maximum(xv[0:8], xv[8:16]),
                    jnp.maximum(xv[16:24], xv[24:32]))
    r = jnp.maximum(r[0:4], r[4:8])
    r = jnp.maximum(r[0:2], r[2:4])
    return jnp.maximum(r[0:1], r[1:2])


def _vsum_tree(xv):
    r = (((xv[0:8] + xv[8:16]) + (xv[16:24] + xv[24:32]))
         + ((xv[32:40] + xv[40:48]) + (xv[48:56] + xv[56:64])))
    r = r[0:4] + r[4:8]
    r = r[0:2] + r[2:4]
    return r[0:1] + r[1:2]


def _body(x_ref, wqkv_ref, bqkv_ref, biast2_ref, wproj_ref, bproj_ref, o_ref,
          qkv_scr, acc_scr):
    xb = x_ref[...]
    qkv_scr[...] = (
        jnp.dot(xb, wqkv_ref[...], preferred_element_type=jnp.float32)
        + bqkv_ref[...])
    acc_scr[...] = jnp.zeros((_T, _DIM), jnp.float32)

    lane = jax.lax.broadcasted_iota(jnp.int32, (_N, 128), 1)

    for g in range(3):
        for cp in range(2):
            c1, c2 = 2 * cp, 2 * cp + 1
            m1 = (lane >= c1 * _HD) & (lane < (c1 + 1) * _HD)
            m2 = (lane >= c2 * _HD) & (lane < (c2 + 1) * _HD)
            biast2_h = biast2_ref[2 * g + cp]
            for w in range(_WB):
                rs = w * _N
                qg = qkv_scr[pl.ds(rs, _N), pl.ds(g * 128, 128)]
                kg = qkv_scr[pl.ds(rs, _N), pl.ds((3 + g) * 128, 128)]
                vg = qkv_scr[pl.ds(rs, _N), pl.ds((6 + g) * 128, 128)]
                qs = qg * _SCALE
                q2 = jnp.concatenate(
                    [jnp.where(m1, qs, 0.0),
                     jnp.where(m2, qs, 0.0)], axis=0)
                s2 = jax.lax.dot_general(q2, kg, (((1,), (1,)), ((), ())),
                                         preferred_element_type=jnp.float32)
                st = jnp.transpose(s2) + biast2_h
                hi = jnp.maximum(st[0:32], st[32:64])
                lo = jnp.minimum(st[0:32], st[32:64])
                m0 = _vmax_tree32(hi)
                m = m0
                for _ in range(_K - 1):
                    msk = hi >= m
                    hi = jnp.where(msk, lo, hi)
                    lo = jnp.where(msk, _NEG, lo)
                    m = _vmax_tree32(hi)
                p = jnp.where(st >= m, jnp.exp(st - m0), 0.0)
                p = p * (1.0 / _vsum_tree(p))
                pt = jnp.transpose(p)
                out2 = jax.lax.dot_general(
                    pt, vg, (((1,), (0,)), ((), ())),
                    preferred_element_type=jnp.float32)
                comb = jnp.where(m1, out2[0:_N],
                                 jnp.where(m2, out2[_N:], 0.0))
                acc_scr[pl.ds(rs, _N), pl.ds(g * 128, 128)] += comb

    o_ref[...] = (
        jnp.dot(acc_scr[...], wproj_ref[...],
                preferred_element_type=jnp.float32)
        + bproj_ref[...])


def kernel(x, Wqkv, bqkv, rpb_table, Wproj, bproj):
    x2 = x.reshape(_B * _N, _DIM)
    rel = _rel_pos_index(_WIN).reshape(-1)
    bias = rpb_table[rel].reshape(_N, _N, _NH)
    biast = jnp.transpose(bias, (2, 1, 0))
    biast2 = biast.reshape(_NH // 2, 2, _N, _N)
    biast2 = jnp.transpose(biast2, (0, 2, 1, 3)).reshape(
        _NH // 2, _N, 2 * _N)

    out = pl.pallas_call(
        _body,
        grid=(_B // _WB,),
        in_specs=[
            pl.BlockSpec((_T, _DIM), lambda i: (i, 0)),
            pl.BlockSpec((_DIM, 3 * _DIM), lambda i: (0, 0)),
            pl.BlockSpec((1, 3 * _DIM), lambda i: (0, 0)),
            pl.BlockSpec((_NH // 2, _N, 2 * _N), lambda i: (0, 0, 0)),
            pl.BlockSpec((_DIM, _DIM), lambda i: (0, 0)),
            pl.BlockSpec((1, _DIM), lambda i: (0, 0)),
        ],
        out_specs=pl.BlockSpec((_T, _DIM), lambda i: (i, 0)),
        out_shape=jax.ShapeDtypeStruct((_B * _N, _DIM), jnp.float32),
        scratch_shapes=[
            pltpu.VMEM((_T, 3 * _DIM), jnp.float32),
            pltpu.VMEM((_T, _DIM), jnp.float32),
        ],
    )(x2, Wqkv, bqkv.reshape(1, -1), biast2, Wproj, bproj.reshape(1, -1))
    return out.reshape(_B, _N, _DIM)
